# baseline (speedup 1.0000x reference)
import functools

import jax
import jax.numpy as jnp
from jax import lax
from jax.experimental import pallas as pl
from jax.experimental.pallas import tpu as pltpu
from jax.experimental.pallas import tpu_sc as plsc

_CHANNELS = 32768
_ROWS = 128
_N = _ROWS * _CHANNELS
_NW = 32
_PER_W = _N // _NW
_ZCH = 65536
_NDMA = _PER_W // _ZCH

_mesh = plsc.VectorSubcoreMesh(core_axis_name="c", subcore_axis_name="s")


@functools.partial(
    pl.kernel,
    out_type=jax.ShapeDtypeStruct((_N,), jnp.float32),
    mesh=_mesh,
    scratch_types=[
        pltpu.VMEM((_ZCH,), jnp.float32),
        pltpu.SemaphoreType.DMA,
    ],
)
def _sc_zeros(out_hbm, zbuf, sem):
    wid = lax.axis_index("s") * 2 + lax.axis_index("c")

    @pl.loop(0, _ZCH // 16)
    def _zero_zbuf(i):
        zbuf[pl.ds(i * 16, 16)] = jnp.zeros((16,), jnp.float32)

    base = wid * _PER_W
    copies = [
        pltpu.async_copy(
            zbuf, out_hbm.at[pl.ds(base + j * _ZCH, _ZCH)], sem
        )
        for j in range(_NDMA)
    ]
    for c in copies:
        c.wait()


def kernel(x):
    del x
    return _sc_zeros().reshape(_ROWS, _CHANNELS)


# speedup vs baseline: 5.1954x; 5.1954x over previous
import jax
import jax.numpy as jnp
from jax.experimental import pallas as pl

_CHANNELS = 32768
_ROWS = 128
_BLOCK_ROWS = 64


def _onehot_argmax_kernel(x_ref, o_ref):
    x = x_ref[...]
    m = jnp.max(x, axis=1, keepdims=True)
    col = jax.lax.broadcasted_iota(jnp.int32, x.shape, 1)
    rev = (_CHANNELS - 1) - col
    ridx = jnp.max(jnp.where(x == m, rev, -1), axis=1, keepdims=True)
    o_ref[...] = (rev == ridx).astype(jnp.float32)


def kernel(x):
    return pl.pallas_call(
        _onehot_argmax_kernel,
        grid=(_ROWS // _BLOCK_ROWS,),
        in_specs=[pl.BlockSpec((_BLOCK_ROWS, _CHANNELS), lambda i: (i, 0))],
        out_specs=pl.BlockSpec((_BLOCK_ROWS, _CHANNELS), lambda i: (i, 0)),
        out_shape=jax.ShapeDtypeStruct((_ROWS, _CHANNELS), jnp.float32),
    )(x)
